# Initial kernel scaffold; baseline (speedup 1.0000x reference)
import jax
import jax.numpy as jnp
from jax.experimental import pallas as pl


def kernel(x, edge_index, gcn_w, gcn_b, sage_wl, sage_bl, sage_wr, gin_w, gin_b, graph_wrel, graph_brel, graph_wroot, fc_w, fc_b):
    raise NotImplementedError("write your pallas kernel here")


# speedup vs baseline: 18.5439x; 18.5439x over previous
import functools
import jax
import jax.numpy as jnp
from jax import lax
from jax.experimental import pallas as pl
from jax.experimental.pallas import tpu as pltpu
from jax.experimental.pallas import tpu_sc as plsc

N = 10000
D = 128
H = 16
OUT = 16

N_PAD = 10240
CHUNK = 128
NUM_TILES = 32
ROWS_PER_TILE = N_PAD // 16


def _tile_id():
    cid = lax.axis_index("c")
    sid = lax.axis_index("s")
    return cid * 16 + sid, cid, sid


def _make_sc_hist(e_pad):
    chunks_per_tile = e_pad // (NUM_TILES * CHUNK)
    edges_per_tile = chunks_per_tile * CHUNK
    mesh = plsc.VectorSubcoreMesh(core_axis_name="c", subcore_axis_name="s")

    @functools.partial(
        pl.kernel,
        out_type=jax.ShapeDtypeStruct((2, N_PAD, 16), jnp.float32),
        mesh=mesh,
        compiler_params=pltpu.CompilerParams(use_tc_tiling_on_sc=False),
        scratch_types=[
            pltpu.VMEM((CHUNK,), jnp.int32),
            pltpu.VMEM((CHUNK, 16), jnp.float32),
            pltpu.VMEM_SHARED((N_PAD, 16), jnp.float32),
        ],
    )
    def hist(dst_hbm, ones_hbm, zeros_hbm, out_hbm, dst_v, ones_v, cnt_sh):
        tid, cid, sid = _tile_id()
        row0 = sid * ROWS_PER_TILE
        pltpu.sync_copy(zeros_hbm.at[pl.ds(row0, ROWS_PER_TILE)],
                        cnt_sh.at[pl.ds(row0, ROWS_PER_TILE)])
        pltpu.sync_copy(ones_hbm, ones_v)
        plsc.subcore_barrier()

        base = tid * edges_per_tile

        def body(j, carry):
            off = base + j * CHUNK
            pltpu.sync_copy(dst_hbm.at[pl.ds(off, CHUNK)], dst_v)
            pltpu.sync_copy(ones_v, cnt_sh.at[dst_v], add=True)
            return carry

        lax.fori_loop(0, chunks_per_tile, body, 0)
        plsc.subcore_barrier()
        pltpu.sync_copy(cnt_sh.at[pl.ds(row0, ROWS_PER_TILE)],
                        out_hbm.at[cid, pl.ds(row0, ROWS_PER_TILE)])

    return hist


def _make_sc_segsum(e_pad):
    chunks_per_tile = e_pad // (NUM_TILES * CHUNK)
    edges_per_tile = chunks_per_tile * CHUNK
    mesh = plsc.VectorSubcoreMesh(core_axis_name="c", subcore_axis_name="s")

    @functools.partial(
        pl.kernel,
        out_type=jax.ShapeDtypeStruct((2, N_PAD, 64), jnp.float32),
        mesh=mesh,
        compiler_params=pltpu.CompilerParams(use_tc_tiling_on_sc=False),
        scratch_types=[
            pltpu.VMEM((CHUNK,), jnp.int32),
            pltpu.VMEM((CHUNK,), jnp.int32),
            pltpu.VMEM((CHUNK, 64), jnp.float32),
            pltpu.VMEM_SHARED((N_PAD, 64), jnp.float32),
            pltpu.SemaphoreType.DMA,
        ],
    )
    def segsum(m_hbm, src_hbm, dst_hbm, zeros_hbm, out_hbm,
               src_v, dst_v, rows_v, s_sh, sem):
        tid, cid, sid = _tile_id()
        row0 = sid * ROWS_PER_TILE
        pltpu.sync_copy(zeros_hbm.at[pl.ds(row0, ROWS_PER_TILE)],
                        s_sh.at[pl.ds(row0, ROWS_PER_TILE)])
        plsc.subcore_barrier()

        base = tid * edges_per_tile

        def body(j, carry):
            off = base + j * CHUNK
            pltpu.sync_copy(src_hbm.at[pl.ds(off, CHUNK)], src_v)
            pltpu.sync_copy(dst_hbm.at[pl.ds(off, CHUNK)], dst_v)
            pltpu.async_copy(m_hbm.at[src_v], rows_v, sem).wait()
            pltpu.sync_copy(rows_v, s_sh.at[dst_v], add=True)
            return carry

        lax.fori_loop(0, chunks_per_tile, body, 0)
        plsc.subcore_barrier()
        pltpu.sync_copy(s_sh.at[pl.ds(row0, ROWS_PER_TILE)],
                        out_hbm.at[cid, pl.ds(row0, ROWS_PER_TILE)])

    return segsum


def _tc_project_body(x_ref, w_ref, c0_ref, c1_ref, m_ref):
    p = jnp.dot(x_ref[...], w_ref[...], preferred_element_type=jnp.float32)
    deg = c0_ref[:, 0:1] + c1_ref[:, 0:1] + 1.0
    dinv = lax.rsqrt(deg)
    col = lax.broadcasted_iota(jnp.int32, p.shape, 1)
    m_ref[...] = jnp.where(col < 16, p * dinv, p)


def _tc_combine_body(x_ref, s0_ref, s1_ref, m_ref, c0_ref, c1_ref,
                     wn_ref, bc_ref, fw_ref, fb_ref, out_ref):
    s = s0_ref[...] + s1_ref[...]
    m = m_ref[...]
    cnt = c0_ref[:, 0:1] + c1_ref[:, 0:1]
    dinv = lax.rsqrt(cnt + 1.0)
    a = jnp.dot(x_ref[...], wn_ref[...], preferred_element_type=jnp.float32)
    bc = bc_ref[...]
    gcn = dinv * (s[:, 0:16] + m[:, 0:16]) + bc[:, 0:16]
    sage = s[:, 16:32] / jnp.maximum(cnt, 1.0) + bc[:, 16:32] + a[:, 0:16]
    gin = m[:, 32:48] + s[:, 32:48] + bc[:, 32:48]
    graph = s[:, 48:64] + bc[:, 48:64] + a[:, 16:32]
    cat = jnp.concatenate(
        [jax.nn.relu(gcn), jax.nn.relu(sage), jax.nn.relu(gin),
         jax.nn.relu(graph)], axis=1)
    out = jnp.dot(cat, fw_ref[...], preferred_element_type=jnp.float32)
    out_ref[...] = jax.nn.sigmoid(out + fb_ref[...])


def kernel(x, edge_index, gcn_w, gcn_b, sage_wl, sage_bl, sage_wr,
           gin_w, gin_b, graph_wrel, graph_brel, graph_wroot, fc_w, fc_b):
    e = edge_index.shape[1]
    e_pad = ((e + NUM_TILES * CHUNK - 1) // (NUM_TILES * CHUNK)) * (NUM_TILES * CHUNK)
    pad_e = e_pad - e
    src = jnp.concatenate([edge_index[0], jnp.full((pad_e,), N, jnp.int32)])
    dst = jnp.concatenate([edge_index[1], jnp.full((pad_e,), N, jnp.int32)])

    x_pad = jnp.pad(x, ((0, N_PAD - N), (0, 0)))
    w_edge = jnp.concatenate([gcn_w, sage_wl, gin_w, graph_wrel], axis=1)
    w_node = jnp.concatenate([sage_wr, graph_wroot], axis=1)
    b_cat = jnp.concatenate([gcn_b, sage_bl, gin_b, graph_brel]).reshape(1, 64)

    ones128 = jnp.ones((CHUNK, 16), jnp.float32)
    zeros16 = jnp.zeros((N_PAD, 16), jnp.float32)
    zeros64 = jnp.zeros((N_PAD, 64), jnp.float32)

    cnt2 = _make_sc_hist(e_pad)(dst, ones128, zeros16)

    blk = 1024
    m = pl.pallas_call(
        _tc_project_body,
        grid=(N_PAD // blk,),
        in_specs=[
            pl.BlockSpec((blk, D), lambda i: (i, 0)),
            pl.BlockSpec((D, 64), lambda i: (0, 0)),
            pl.BlockSpec((blk, 16), lambda i: (i, 0)),
            pl.BlockSpec((blk, 16), lambda i: (i, 0)),
        ],
        out_specs=pl.BlockSpec((blk, 64), lambda i: (i, 0)),
        out_shape=jax.ShapeDtypeStruct((N_PAD, 64), jnp.float32),
    )(x_pad, w_edge, cnt2[0], cnt2[1])

    s2 = _make_sc_segsum(e_pad)(m, src, dst, zeros64)

    blk2 = 2000
    out = pl.pallas_call(
        _tc_combine_body,
        grid=(N // blk2,),
        in_specs=[
            pl.BlockSpec((blk2, D), lambda i: (i, 0)),
            pl.BlockSpec((blk2, 64), lambda i: (i, 0)),
            pl.BlockSpec((blk2, 64), lambda i: (i, 0)),
            pl.BlockSpec((blk2, 64), lambda i: (i, 0)),
            pl.BlockSpec((blk2, 16), lambda i: (i, 0)),
            pl.BlockSpec((blk2, 16), lambda i: (i, 0)),
            pl.BlockSpec((D, 32), lambda i: (0, 0)),
            pl.BlockSpec((1, 64), lambda i: (0, 0)),
            pl.BlockSpec((64, OUT), lambda i: (0, 0)),
            pl.BlockSpec((1, OUT), lambda i: (0, 0)),
        ],
        out_specs=pl.BlockSpec((blk2, OUT), lambda i: (i, 0)),
        out_shape=jax.ShapeDtypeStruct((N, OUT), jnp.float32),
    )(x, s2[0], s2[1], m, cnt2[0], cnt2[1], w_node, b_cat, fc_w,
      fc_b.reshape(1, OUT))
    return out


# speedup vs baseline: 21.0879x; 1.1372x over previous
import functools
import jax
import jax.numpy as jnp
from jax import lax
from jax.experimental import pallas as pl
from jax.experimental.pallas import tpu as pltpu
from jax.experimental.pallas import tpu_sc as plsc

N = 10000
D = 128
H = 16
OUT = 16

N_PAD = 10240
CHUNK = 128
NUM_TILES = 32
ROWS_PER_TILE = N_PAD // 16
K = 4
NBUF = 2


def _tile_id():
    cid = lax.axis_index("c")
    sid = lax.axis_index("s")
    return cid * 16 + sid, cid, sid


def _make_sc_hist(e_pad):
    chunks_per_tile = e_pad // (NUM_TILES * CHUNK)
    edges_per_tile = chunks_per_tile * CHUNK
    mesh = plsc.VectorSubcoreMesh(core_axis_name="c", subcore_axis_name="s")

    @functools.partial(
        pl.kernel,
        out_type=jax.ShapeDtypeStruct((2, N_PAD, 16), jnp.float32),
        mesh=mesh,
        compiler_params=pltpu.CompilerParams(use_tc_tiling_on_sc=False),
        scratch_types=[
            pltpu.VMEM((CHUNK,), jnp.int32),
            pltpu.VMEM((CHUNK, 16), jnp.float32),
            pltpu.VMEM_SHARED((N_PAD, 16), jnp.float32),
        ],
    )
    def hist(dst_hbm, ones_hbm, zeros_hbm, out_hbm, dst_v, ones_v, cnt_sh):
        tid, cid, sid = _tile_id()
        row0 = sid * ROWS_PER_TILE
        pltpu.sync_copy(zeros_hbm.at[pl.ds(row0, ROWS_PER_TILE)],
                        cnt_sh.at[pl.ds(row0, ROWS_PER_TILE)])
        pltpu.sync_copy(ones_hbm, ones_v)
        plsc.subcore_barrier()

        base = tid * edges_per_tile

        def body(j, carry):
            off = base + j * CHUNK
            pltpu.sync_copy(dst_hbm.at[pl.ds(off, CHUNK)], dst_v)
            pltpu.sync_copy(ones_v, cnt_sh.at[dst_v], add=True)
            return carry

        lax.fori_loop(0, chunks_per_tile, body, 0)
        plsc.subcore_barrier()
        pltpu.sync_copy(cnt_sh.at[pl.ds(row0, ROWS_PER_TILE)],
                        out_hbm.at[cid, pl.ds(row0, ROWS_PER_TILE)])

    return hist


def _make_sc_segsum(e_pad):
    chunks_per_tile = e_pad // (NUM_TILES * CHUNK)
    mesh = plsc.VectorSubcoreMesh(core_axis_name="c", subcore_axis_name="s")

    @functools.partial(
        pl.kernel,
        out_type=jax.ShapeDtypeStruct((2, N_PAD, 64), jnp.float32),
        mesh=mesh,
        compiler_params=pltpu.CompilerParams(use_tc_tiling_on_sc=False),
        scratch_types=[
            pltpu.VMEM((2, CHUNK), jnp.int32),
            pltpu.VMEM((CHUNK, 64), jnp.float32),
            pltpu.VMEM_SHARED((N_PAD, 64), jnp.float32),
        ],
    )
    def segsum(m_hbm, idx_hbm, zeros_hbm, out_hbm, idx_v, rows_v, s_sh):
        tid, cid, sid = _tile_id()
        row0 = sid * ROWS_PER_TILE
        pltpu.sync_copy(zeros_hbm.at[pl.ds(row0, ROWS_PER_TILE)],
                        s_sh.at[pl.ds(row0, ROWS_PER_TILE)])
        plsc.subcore_barrier()

        base_c = tid * chunks_per_tile

        def body(j, carry):
            pltpu.sync_copy(idx_hbm.at[base_c + j], idx_v)
            pltpu.sync_copy(m_hbm.at[idx_v.at[0]], rows_v)
            pltpu.sync_copy(rows_v, s_sh.at[idx_v.at[1]], add=True)
            return carry

        lax.fori_loop(0, chunks_per_tile, body, 0)
        plsc.subcore_barrier()
        pltpu.sync_copy(s_sh.at[pl.ds(row0, ROWS_PER_TILE)],
                        out_hbm.at[cid, pl.ds(row0, ROWS_PER_TILE)])

    return segsum


def _tc_project_body(x_ref, w_ref, c0_ref, c1_ref, m_ref):
    p = jnp.dot(x_ref[...], w_ref[...], preferred_element_type=jnp.float32)
    deg = c0_ref[:, 0:1] + c1_ref[:, 0:1] + 1.0
    dinv = lax.rsqrt(deg)
    col = lax.broadcasted_iota(jnp.int32, p.shape, 1)
    m_ref[...] = jnp.where(col < 16, p * dinv, p)


def _tc_combine_body(x_ref, s0_ref, s1_ref, m_ref, c0_ref, c1_ref,
                     wn_ref, bc_ref, fw_ref, fb_ref, out_ref):
    s = s0_ref[...] + s1_ref[...]
    m = m_ref[...]
    cnt = c0_ref[:, 0:1] + c1_ref[:, 0:1]
    dinv = lax.rsqrt(cnt + 1.0)
    a = jnp.dot(x_ref[...], wn_ref[...], preferred_element_type=jnp.float32)
    bc = bc_ref[...]
    gcn = dinv * (s[:, 0:16] + m[:, 0:16]) + bc[:, 0:16]
    sage = s[:, 16:32] / jnp.maximum(cnt, 1.0) + bc[:, 16:32] + a[:, 0:16]
    gin = m[:, 32:48] + s[:, 32:48] + bc[:, 32:48]
    graph = s[:, 48:64] + bc[:, 48:64] + a[:, 16:32]
    cat = jnp.concatenate(
        [jax.nn.relu(gcn), jax.nn.relu(sage), jax.nn.relu(gin),
         jax.nn.relu(graph)], axis=1)
    out = jnp.dot(cat, fw_ref[...], preferred_element_type=jnp.float32)
    out_ref[...] = jax.nn.sigmoid(out + fb_ref[...])


def kernel(x, edge_index, gcn_w, gcn_b, sage_wl, sage_bl, sage_wr,
           gin_w, gin_b, graph_wrel, graph_brel, graph_wroot, fc_w, fc_b):
    e = edge_index.shape[1]
    e_pad = ((e + NUM_TILES * CHUNK - 1) // (NUM_TILES * CHUNK)) * (NUM_TILES * CHUNK)
    pad_e = e_pad - e
    src = jnp.concatenate([edge_index[0], jnp.full((pad_e,), N, jnp.int32)])
    dst = jnp.concatenate([edge_index[1], jnp.full((pad_e,), N, jnp.int32)])

    x_pad = jnp.pad(x, ((0, N_PAD - N), (0, 0)))
    w_edge = jnp.concatenate([gcn_w, sage_wl, gin_w, graph_wrel], axis=1)
    w_node = jnp.concatenate([sage_wr, graph_wroot], axis=1)
    b_cat = jnp.concatenate([gcn_b, sage_bl, gin_b, graph_brel]).reshape(1, 64)

    ones128 = jnp.ones((CHUNK, 16), jnp.float32)
    zeros16 = jnp.zeros((N_PAD, 16), jnp.float32)
    zeros64 = jnp.zeros((N_PAD, 64), jnp.float32)

    cnt2 = _make_sc_hist(e_pad)(dst, ones128, zeros16)

    blk = 1024
    m = pl.pallas_call(
        _tc_project_body,
        grid=(N_PAD // blk,),
        in_specs=[
            pl.BlockSpec((blk, D), lambda i: (i, 0)),
            pl.BlockSpec((D, 64), lambda i: (0, 0)),
            pl.BlockSpec((blk, 16), lambda i: (i, 0)),
            pl.BlockSpec((blk, 16), lambda i: (i, 0)),
        ],
        out_specs=pl.BlockSpec((blk, 64), lambda i: (i, 0)),
        out_shape=jax.ShapeDtypeStruct((N_PAD, 64), jnp.float32),
    )(x_pad, w_edge, cnt2[0], cnt2[1])

    idx = jnp.stack([src.reshape(-1, CHUNK), dst.reshape(-1, CHUNK)], axis=1)
    s2 = _make_sc_segsum(e_pad)(m, idx, zeros64)

    blk2 = 2000
    out = pl.pallas_call(
        _tc_combine_body,
        grid=(N // blk2,),
        in_specs=[
            pl.BlockSpec((blk2, D), lambda i: (i, 0)),
            pl.BlockSpec((blk2, 64), lambda i: (i, 0)),
            pl.BlockSpec((blk2, 64), lambda i: (i, 0)),
            pl.BlockSpec((blk2, 64), lambda i: (i, 0)),
            pl.BlockSpec((blk2, 16), lambda i: (i, 0)),
            pl.BlockSpec((blk2, 16), lambda i: (i, 0)),
            pl.BlockSpec((D, 32), lambda i: (0, 0)),
            pl.BlockSpec((1, 64), lambda i: (0, 0)),
            pl.BlockSpec((64, OUT), lambda i: (0, 0)),
            pl.BlockSpec((1, OUT), lambda i: (0, 0)),
        ],
        out_specs=pl.BlockSpec((blk2, OUT), lambda i: (i, 0)),
        out_shape=jax.ShapeDtypeStruct((N, OUT), jnp.float32),
    )(x, s2[0], s2[1], m, cnt2[0], cnt2[1], w_node, b_cat, fc_w,
      fc_b.reshape(1, OUT))
    return out


# speedup vs baseline: 25.9587x; 1.2310x over previous
import functools
import jax
import jax.numpy as jnp
from jax import lax
from jax.experimental import pallas as pl
from jax.experimental.pallas import tpu as pltpu
from jax.experimental.pallas import tpu_sc as plsc

N = 10000
D = 128
H = 16
OUT = 16

N_PAD = 10240
CHUNK = 128
NUM_TILES = 32
ROWS_PER_TILE = N_PAD // 16
K = 4
NBUF = 2


def _tile_id():
    cid = lax.axis_index("c")
    sid = lax.axis_index("s")
    return cid * 16 + sid, cid, sid


def _make_sc_hist(e_pad):
    chunks_per_tile = e_pad // (NUM_TILES * CHUNK)
    edges_per_tile = chunks_per_tile * CHUNK
    mesh = plsc.VectorSubcoreMesh(core_axis_name="c", subcore_axis_name="s")

    @functools.partial(
        pl.kernel,
        out_type=jax.ShapeDtypeStruct((2, N_PAD, 16), jnp.float32),
        mesh=mesh,
        compiler_params=pltpu.CompilerParams(use_tc_tiling_on_sc=False),
        scratch_types=[
            pltpu.VMEM((e_pad // (NUM_TILES * CHUNK), CHUNK), jnp.int32),
            pltpu.VMEM((CHUNK, 16), jnp.float32),
            pltpu.VMEM_SHARED((N_PAD, 16), jnp.float32),
        ],
    )
    def hist(dst_hbm, ones_hbm, zeros_hbm, out_hbm, dst_v, ones_v, cnt_sh):
        tid, cid, sid = _tile_id()
        row0 = sid * ROWS_PER_TILE
        pltpu.sync_copy(zeros_hbm.at[pl.ds(row0, ROWS_PER_TILE)],
                        cnt_sh.at[pl.ds(row0, ROWS_PER_TILE)])
        pltpu.sync_copy(ones_hbm, ones_v)
        pltpu.sync_copy(dst_hbm.at[pl.ds(tid * chunks_per_tile, chunks_per_tile)],
                        dst_v)
        plsc.subcore_barrier()

        def body(j, carry):
            pltpu.sync_copy(ones_v, cnt_sh.at[dst_v.at[j]], add=True)
            return carry

        lax.fori_loop(0, chunks_per_tile, body, 0)
        plsc.subcore_barrier()
        pltpu.sync_copy(cnt_sh.at[pl.ds(row0, ROWS_PER_TILE)],
                        out_hbm.at[cid, pl.ds(row0, ROWS_PER_TILE)])

    return hist


def _make_sc_segsum(e_pad):
    chunks_per_tile = e_pad // (NUM_TILES * CHUNK)
    mesh = plsc.VectorSubcoreMesh(core_axis_name="c", subcore_axis_name="s")

    @functools.partial(
        pl.kernel,
        out_type=jax.ShapeDtypeStruct((2, N_PAD, 64), jnp.float32),
        mesh=mesh,
        compiler_params=pltpu.CompilerParams(use_tc_tiling_on_sc=False),
        scratch_types=[
            pltpu.VMEM((chunks_per_tile, 2, CHUNK), jnp.int32),
            pltpu.VMEM((CHUNK, 64), jnp.float32),
            pltpu.VMEM_SHARED((N_PAD, 64), jnp.float32),
        ],
    )
    def segsum(m_hbm, idx_hbm, zeros_hbm, out_hbm, idx_v, rows_v, s_sh):
        tid, cid, sid = _tile_id()
        row0 = sid * ROWS_PER_TILE
        pltpu.sync_copy(zeros_hbm.at[pl.ds(row0, ROWS_PER_TILE)],
                        s_sh.at[pl.ds(row0, ROWS_PER_TILE)])
        pltpu.sync_copy(idx_hbm.at[pl.ds(tid * chunks_per_tile, chunks_per_tile)],
                        idx_v)
        plsc.subcore_barrier()

        def body(j, carry):
            pltpu.sync_copy(m_hbm.at[idx_v.at[j, 0]], rows_v)
            pltpu.sync_copy(rows_v, s_sh.at[idx_v.at[j, 1]], add=True)
            return carry

        lax.fori_loop(0, chunks_per_tile, body, 0)
        plsc.subcore_barrier()
        pltpu.sync_copy(s_sh.at[pl.ds(row0, ROWS_PER_TILE)],
                        out_hbm.at[cid, pl.ds(row0, ROWS_PER_TILE)])

    return segsum


def _tc_project_body(x_ref, w_ref, c0_ref, c1_ref, m_ref):
    p = jnp.dot(x_ref[...], w_ref[...], preferred_element_type=jnp.float32)
    deg = c0_ref[:, 0:1] + c1_ref[:, 0:1] + 1.0
    dinv = lax.rsqrt(deg)
    col = lax.broadcasted_iota(jnp.int32, p.shape, 1)
    m_ref[...] = jnp.where(col < 16, p * dinv, p)


def _tc_combine_body(x_ref, s0_ref, s1_ref, m_ref, c0_ref, c1_ref,
                     wn_ref, bc_ref, fw_ref, fb_ref, out_ref):
    s = s0_ref[...] + s1_ref[...]
    m = m_ref[...]
    cnt = c0_ref[:, 0:1] + c1_ref[:, 0:1]
    dinv = lax.rsqrt(cnt + 1.0)
    a = jnp.dot(x_ref[...], wn_ref[...], preferred_element_type=jnp.float32)
    bc = bc_ref[...]
    gcn = dinv * (s[:, 0:16] + m[:, 0:16]) + bc[:, 0:16]
    sage = s[:, 16:32] / jnp.maximum(cnt, 1.0) + bc[:, 16:32] + a[:, 0:16]
    gin = m[:, 32:48] + s[:, 32:48] + bc[:, 32:48]
    graph = s[:, 48:64] + bc[:, 48:64] + a[:, 16:32]
    cat = jnp.concatenate(
        [jax.nn.relu(gcn), jax.nn.relu(sage), jax.nn.relu(gin),
         jax.nn.relu(graph)], axis=1)
    out = jnp.dot(cat, fw_ref[...], preferred_element_type=jnp.float32)
    out_ref[...] = jax.nn.sigmoid(out + fb_ref[...])


def kernel(x, edge_index, gcn_w, gcn_b, sage_wl, sage_bl, sage_wr,
           gin_w, gin_b, graph_wrel, graph_brel, graph_wroot, fc_w, fc_b):
    e = edge_index.shape[1]
    e_pad = ((e + NUM_TILES * CHUNK - 1) // (NUM_TILES * CHUNK)) * (NUM_TILES * CHUNK)
    pad_e = e_pad - e
    src = jnp.concatenate([edge_index[0], jnp.full((pad_e,), N, jnp.int32)])
    dst = jnp.concatenate([edge_index[1], jnp.full((pad_e,), N, jnp.int32)])

    x_pad = jnp.pad(x, ((0, N_PAD - N), (0, 0)))
    w_edge = jnp.concatenate([gcn_w, sage_wl, gin_w, graph_wrel], axis=1)
    w_node = jnp.concatenate([sage_wr, graph_wroot], axis=1)
    b_cat = jnp.concatenate([gcn_b, sage_bl, gin_b, graph_brel]).reshape(1, 64)

    ones128 = jnp.ones((CHUNK, 16), jnp.float32)
    zeros16 = jnp.zeros((N_PAD, 16), jnp.float32)
    zeros64 = jnp.zeros((N_PAD, 64), jnp.float32)

    cnt2 = _make_sc_hist(e_pad)(dst.reshape(-1, CHUNK), ones128, zeros16)

    blk = 1024
    m = pl.pallas_call(
        _tc_project_body,
        grid=(N_PAD // blk,),
        in_specs=[
            pl.BlockSpec((blk, D), lambda i: (i, 0)),
            pl.BlockSpec((D, 64), lambda i: (0, 0)),
            pl.BlockSpec((blk, 16), lambda i: (i, 0)),
            pl.BlockSpec((blk, 16), lambda i: (i, 0)),
        ],
        out_specs=pl.BlockSpec((blk, 64), lambda i: (i, 0)),
        out_shape=jax.ShapeDtypeStruct((N_PAD, 64), jnp.float32),
    )(x_pad, w_edge, cnt2[0], cnt2[1])

    idx = jnp.stack([src.reshape(-1, CHUNK), dst.reshape(-1, CHUNK)], axis=1)
    s2 = _make_sc_segsum(e_pad)(m, idx, zeros64)

    blk2 = 2000
    out = pl.pallas_call(
        _tc_combine_body,
        grid=(N // blk2,),
        in_specs=[
            pl.BlockSpec((blk2, D), lambda i: (i, 0)),
            pl.BlockSpec((blk2, 64), lambda i: (i, 0)),
            pl.BlockSpec((blk2, 64), lambda i: (i, 0)),
            pl.BlockSpec((blk2, 64), lambda i: (i, 0)),
            pl.BlockSpec((blk2, 16), lambda i: (i, 0)),
            pl.BlockSpec((blk2, 16), lambda i: (i, 0)),
            pl.BlockSpec((D, 32), lambda i: (0, 0)),
            pl.BlockSpec((1, 64), lambda i: (0, 0)),
            pl.BlockSpec((64, OUT), lambda i: (0, 0)),
            pl.BlockSpec((1, OUT), lambda i: (0, 0)),
        ],
        out_specs=pl.BlockSpec((blk2, OUT), lambda i: (i, 0)),
        out_shape=jax.ShapeDtypeStruct((N, OUT), jnp.float32),
    )(x, s2[0], s2[1], m, cnt2[0], cnt2[1], w_node, b_cat, fc_w,
      fc_b.reshape(1, OUT))
    return out


# speedup vs baseline: 34.3393x; 1.3228x over previous
import functools
import jax
import jax.numpy as jnp
from jax import lax
from jax.experimental import pallas as pl
from jax.experimental.pallas import tpu as pltpu
from jax.experimental.pallas import tpu_sc as plsc

N = 10000
D = 128
H = 16
OUT = 16

N_PAD = 10240
CHUNK = 128
NUM_TILES = 32
ROWS_PER_TILE = N_PAD // 16
K = 4
NBUF = 2


def _tile_id():
    cid = lax.axis_index("c")
    sid = lax.axis_index("s")
    return cid * 16 + sid, cid, sid


def _make_sc_hist(e_pad):
    chunks_per_tile = e_pad // (NUM_TILES * CHUNK)
    edges_per_tile = chunks_per_tile * CHUNK
    mesh = plsc.VectorSubcoreMesh(core_axis_name="c", subcore_axis_name="s")

    @functools.partial(
        pl.kernel,
        out_type=jax.ShapeDtypeStruct((2, N_PAD, 16), jnp.float32),
        mesh=mesh,
        compiler_params=pltpu.CompilerParams(use_tc_tiling_on_sc=False),
        scratch_types=[
            pltpu.VMEM((e_pad // (NUM_TILES * CHUNK), CHUNK), jnp.int32),
            pltpu.VMEM((CHUNK, 16), jnp.float32),
            pltpu.VMEM_SHARED((N_PAD, 16), jnp.float32),
        ],
    )
    def hist(dst_hbm, ones_hbm, zeros_hbm, out_hbm, dst_v, ones_v, cnt_sh):
        tid, cid, sid = _tile_id()
        row0 = sid * ROWS_PER_TILE
        pltpu.sync_copy(zeros_hbm.at[pl.ds(row0, ROWS_PER_TILE)],
                        cnt_sh.at[pl.ds(row0, ROWS_PER_TILE)])
        pltpu.sync_copy(ones_hbm, ones_v)
        pltpu.sync_copy(dst_hbm.at[pl.ds(tid * chunks_per_tile, chunks_per_tile)],
                        dst_v)
        plsc.subcore_barrier()

        def body(j, carry):
            pltpu.sync_copy(ones_v, cnt_sh.at[dst_v.at[j]], add=True)
            return carry

        lax.fori_loop(0, chunks_per_tile, body, 0)
        plsc.subcore_barrier()
        pltpu.sync_copy(cnt_sh.at[pl.ds(row0, ROWS_PER_TILE)],
                        out_hbm.at[cid, pl.ds(row0, ROWS_PER_TILE)])

    return hist


def _make_sc_segsum(e_pad):
    chunks_per_tile = e_pad // (NUM_TILES * CHUNK)
    mesh = plsc.VectorSubcoreMesh(core_axis_name="c", subcore_axis_name="s")

    @functools.partial(
        pl.kernel,
        out_type=jax.ShapeDtypeStruct((2, N_PAD, 64), jnp.float32),
        mesh=mesh,
        compiler_params=pltpu.CompilerParams(use_tc_tiling_on_sc=False),
        scratch_types=[
            pltpu.VMEM((chunks_per_tile, 2, CHUNK), jnp.int32),
            pltpu.VMEM((CHUNK, 64), jnp.float32),
            pltpu.VMEM_SHARED((N_PAD, 64), jnp.float32),
            pltpu.VMEM_SHARED((N_PAD, 64), jnp.float32),
        ],
    )
    def segsum(m_hbm, idx_hbm, zeros_hbm, out_hbm, idx_v, rows_v, m_sh, s_sh):
        tid, cid, sid = _tile_id()
        row0 = sid * ROWS_PER_TILE
        pltpu.sync_copy(zeros_hbm.at[pl.ds(row0, ROWS_PER_TILE)],
                        s_sh.at[pl.ds(row0, ROWS_PER_TILE)])
        pltpu.sync_copy(m_hbm.at[pl.ds(row0, ROWS_PER_TILE)],
                        m_sh.at[pl.ds(row0, ROWS_PER_TILE)])
        pltpu.sync_copy(idx_hbm.at[pl.ds(tid * chunks_per_tile, chunks_per_tile)],
                        idx_v)
        plsc.subcore_barrier()

        def body(j, carry):
            pltpu.sync_copy(m_sh.at[idx_v.at[j, 0]], rows_v)
            pltpu.sync_copy(rows_v, s_sh.at[idx_v.at[j, 1]], add=True)
            return carry

        lax.fori_loop(0, chunks_per_tile, body, 0)
        plsc.subcore_barrier()
        pltpu.sync_copy(s_sh.at[pl.ds(row0, ROWS_PER_TILE)],
                        out_hbm.at[cid, pl.ds(row0, ROWS_PER_TILE)])

    return segsum


def _tc_project_body(x_ref, w_ref, c0_ref, c1_ref, m_ref):
    p = jnp.dot(x_ref[...], w_ref[...], preferred_element_type=jnp.float32)
    deg = c0_ref[:, 0:1] + c1_ref[:, 0:1] + 1.0
    dinv = lax.rsqrt(deg)
    col = lax.broadcasted_iota(jnp.int32, p.shape, 1)
    m_ref[...] = jnp.where(col < 16, p * dinv, p)


def _tc_combine_body(x_ref, s0_ref, s1_ref, m_ref, c0_ref, c1_ref,
                     wn_ref, bc_ref, fw_ref, fb_ref, out_ref):
    s = s0_ref[...] + s1_ref[...]
    m = m_ref[...]
    cnt = c0_ref[:, 0:1] + c1_ref[:, 0:1]
    dinv = lax.rsqrt(cnt + 1.0)
    a = jnp.dot(x_ref[...], wn_ref[...], preferred_element_type=jnp.float32)
    bc = bc_ref[...]
    gcn = dinv * (s[:, 0:16] + m[:, 0:16]) + bc[:, 0:16]
    sage = s[:, 16:32] / jnp.maximum(cnt, 1.0) + bc[:, 16:32] + a[:, 0:16]
    gin = m[:, 32:48] + s[:, 32:48] + bc[:, 32:48]
    graph = s[:, 48:64] + bc[:, 48:64] + a[:, 16:32]
    cat = jnp.concatenate(
        [jax.nn.relu(gcn), jax.nn.relu(sage), jax.nn.relu(gin),
         jax.nn.relu(graph)], axis=1)
    out = jnp.dot(cat, fw_ref[...], preferred_element_type=jnp.float32)
    out_ref[...] = jax.nn.sigmoid(out + fb_ref[...])


def kernel(x, edge_index, gcn_w, gcn_b, sage_wl, sage_bl, sage_wr,
           gin_w, gin_b, graph_wrel, graph_brel, graph_wroot, fc_w, fc_b):
    e = edge_index.shape[1]
    e_pad = ((e + NUM_TILES * CHUNK - 1) // (NUM_TILES * CHUNK)) * (NUM_TILES * CHUNK)
    pad_e = e_pad - e
    pad_rows = N + jnp.arange(pad_e, dtype=jnp.int32) % (N_PAD - N)
    src = jnp.concatenate([edge_index[0], pad_rows])
    dst = jnp.concatenate([edge_index[1], pad_rows])

    x_pad = jnp.pad(x, ((0, N_PAD - N), (0, 0)))
    w_edge = jnp.concatenate([gcn_w, sage_wl, gin_w, graph_wrel], axis=1)
    w_node = jnp.concatenate([sage_wr, graph_wroot], axis=1)
    b_cat = jnp.concatenate([gcn_b, sage_bl, gin_b, graph_brel]).reshape(1, 64)

    ones128 = jnp.ones((CHUNK, 16), jnp.float32)
    zeros16 = jnp.zeros((N_PAD, 16), jnp.float32)
    zeros64 = jnp.zeros((N_PAD, 64), jnp.float32)

    cnt2 = _make_sc_hist(e_pad)(dst.reshape(-1, CHUNK), ones128, zeros16)

    blk = 1024
    m = pl.pallas_call(
        _tc_project_body,
        grid=(N_PAD // blk,),
        in_specs=[
            pl.BlockSpec((blk, D), lambda i: (i, 0)),
            pl.BlockSpec((D, 64), lambda i: (0, 0)),
            pl.BlockSpec((blk, 16), lambda i: (i, 0)),
            pl.BlockSpec((blk, 16), lambda i: (i, 0)),
        ],
        out_specs=pl.BlockSpec((blk, 64), lambda i: (i, 0)),
        out_shape=jax.ShapeDtypeStruct((N_PAD, 64), jnp.float32),
    )(x_pad, w_edge, cnt2[0], cnt2[1])

    idx = jnp.stack([src.reshape(-1, CHUNK), dst.reshape(-1, CHUNK)], axis=1)
    s2 = _make_sc_segsum(e_pad)(m, idx, zeros64)

    blk2 = 2000
    out = pl.pallas_call(
        _tc_combine_body,
        grid=(N // blk2,),
        in_specs=[
            pl.BlockSpec((blk2, D), lambda i: (i, 0)),
            pl.BlockSpec((blk2, 64), lambda i: (i, 0)),
            pl.BlockSpec((blk2, 64), lambda i: (i, 0)),
            pl.BlockSpec((blk2, 64), lambda i: (i, 0)),
            pl.BlockSpec((blk2, 16), lambda i: (i, 0)),
            pl.BlockSpec((blk2, 16), lambda i: (i, 0)),
            pl.BlockSpec((D, 32), lambda i: (0, 0)),
            pl.BlockSpec((1, 64), lambda i: (0, 0)),
            pl.BlockSpec((64, OUT), lambda i: (0, 0)),
            pl.BlockSpec((1, OUT), lambda i: (0, 0)),
        ],
        out_specs=pl.BlockSpec((blk2, OUT), lambda i: (i, 0)),
        out_shape=jax.ShapeDtypeStruct((N, OUT), jnp.float32),
    )(x, s2[0], s2[1], m, cnt2[0], cnt2[1], w_node, b_cat, fc_w,
      fc_b.reshape(1, OUT))
    return out


# speedup vs baseline: 36.6117x; 1.0662x over previous
import functools
import jax
import jax.numpy as jnp
from jax import lax
from jax.experimental import pallas as pl
from jax.experimental.pallas import tpu as pltpu
from jax.experimental.pallas import tpu_sc as plsc

N = 10000
D = 128
H = 16
OUT = 16

N_PAD = 10240
CHUNK = 128
NUM_TILES = 32
ROWS_PER_TILE = N_PAD // 16
K = 4
NBUF = 2


def _tile_id():
    cid = lax.axis_index("c")
    sid = lax.axis_index("s")
    return cid * 16 + sid, cid, sid


def _make_sc_hist(e_pad):
    chunks_per_tile = e_pad // (NUM_TILES * CHUNK)
    mesh = plsc.VectorSubcoreMesh(core_axis_name="c", subcore_axis_name="s")

    @functools.partial(
        pl.kernel,
        out_type=jax.ShapeDtypeStruct((2, N_PAD, 16), jnp.float32),
        mesh=mesh,
        compiler_params=pltpu.CompilerParams(use_tc_tiling_on_sc=False),
        scratch_types=[
            pltpu.VMEM((e_pad // (NUM_TILES * CHUNK), 2, CHUNK), jnp.int32),
            pltpu.VMEM((CHUNK, 16), jnp.float32),
            pltpu.VMEM_SHARED((N_PAD, 16), jnp.float32),
        ],
    )
    def hist(idx_hbm, ones_hbm, zeros_hbm, out_hbm, idx_v, ones_v, cnt_sh):
        tid, cid, sid = _tile_id()
        row0 = sid * ROWS_PER_TILE
        pltpu.sync_copy(zeros_hbm.at[pl.ds(row0, ROWS_PER_TILE)],
                        cnt_sh.at[pl.ds(row0, ROWS_PER_TILE)])
        pltpu.sync_copy(ones_hbm, ones_v)
        pltpu.sync_copy(idx_hbm.at[pl.ds(tid * chunks_per_tile, chunks_per_tile)],
                        idx_v)
        plsc.subcore_barrier()

        def body(j, carry):
            pltpu.sync_copy(ones_v, cnt_sh.at[idx_v.at[j, 1]], add=True)
            return carry

        lax.fori_loop(0, chunks_per_tile, body, 0)
        plsc.subcore_barrier()
        pltpu.sync_copy(cnt_sh.at[pl.ds(row0, ROWS_PER_TILE)],
                        out_hbm.at[cid, pl.ds(row0, ROWS_PER_TILE)])

    return hist


def _make_sc_segsum(e_pad):
    chunks_per_tile = e_pad // (NUM_TILES * CHUNK)
    mesh = plsc.VectorSubcoreMesh(core_axis_name="c", subcore_axis_name="s")

    @functools.partial(
        pl.kernel,
        out_type=jax.ShapeDtypeStruct((2, N_PAD, 64), jnp.float32),
        mesh=mesh,
        compiler_params=pltpu.CompilerParams(use_tc_tiling_on_sc=False),
        scratch_types=[
            pltpu.VMEM((chunks_per_tile, 2, CHUNK), jnp.int32),
            pltpu.VMEM((CHUNK, 64), jnp.float32),
            pltpu.VMEM_SHARED((N_PAD, 64), jnp.float32),
            pltpu.VMEM_SHARED((N_PAD, 64), jnp.float32),
        ],
    )
    def segsum(m_hbm, idx_hbm, zeros_hbm, out_hbm, idx_v, rows_v, m_sh, s_sh):
        tid, cid, sid = _tile_id()
        row0 = sid * ROWS_PER_TILE
        pltpu.sync_copy(zeros_hbm.at[pl.ds(row0, ROWS_PER_TILE)],
                        s_sh.at[pl.ds(row0, ROWS_PER_TILE)])
        pltpu.sync_copy(m_hbm.at[pl.ds(row0, ROWS_PER_TILE)],
                        m_sh.at[pl.ds(row0, ROWS_PER_TILE)])
        pltpu.sync_copy(idx_hbm.at[pl.ds(tid * chunks_per_tile, chunks_per_tile)],
                        idx_v)
        plsc.subcore_barrier()

        def body(j, carry):
            pltpu.sync_copy(m_sh.at[idx_v.at[j, 0]], rows_v)
            pltpu.sync_copy(rows_v, s_sh.at[idx_v.at[j, 1]], add=True)
            return carry

        lax.fori_loop(0, chunks_per_tile, body, 0)
        plsc.subcore_barrier()
        pltpu.sync_copy(s_sh.at[pl.ds(row0, ROWS_PER_TILE)],
                        out_hbm.at[cid, pl.ds(row0, ROWS_PER_TILE)])

    return segsum


def _tc_project_body(x_ref, w_ref, c0_ref, c1_ref, m_ref):
    p = jnp.dot(x_ref[...], w_ref[...], preferred_element_type=jnp.float32)
    deg = c0_ref[0, :, 0:1] + c1_ref[0, :, 0:1] + 1.0
    dinv = lax.rsqrt(deg)
    col = lax.broadcasted_iota(jnp.int32, p.shape, 1)
    m_ref[...] = jnp.where(col < 16, p * dinv, p)


def _tc_combine_body(x_ref, s0_ref, s1_ref, m_ref, c0_ref, c1_ref,
                     wn_ref, bc_ref, fw_ref, fb_ref, out_ref):
    s = s0_ref[0] + s1_ref[0]
    m = m_ref[...]
    cnt = c0_ref[0, :, 0:1] + c1_ref[0, :, 0:1]
    dinv = lax.rsqrt(cnt + 1.0)
    a = jnp.dot(x_ref[...], wn_ref[...], preferred_element_type=jnp.float32)
    bc = bc_ref[...]
    gcn = dinv * (s[:, 0:16] + m[:, 0:16]) + bc[:, 0:16]
    sage = s[:, 16:32] / jnp.maximum(cnt, 1.0) + bc[:, 16:32] + a[:, 0:16]
    gin = m[:, 32:48] + s[:, 32:48] + bc[:, 32:48]
    graph = s[:, 48:64] + bc[:, 48:64] + a[:, 16:32]
    cat = jnp.concatenate(
        [jax.nn.relu(gcn), jax.nn.relu(sage), jax.nn.relu(gin),
         jax.nn.relu(graph)], axis=1)
    out = jnp.dot(cat, fw_ref[...], preferred_element_type=jnp.float32)
    out_ref[...] = jax.nn.sigmoid(out + fb_ref[...])


def kernel(x, edge_index, gcn_w, gcn_b, sage_wl, sage_bl, sage_wr,
           gin_w, gin_b, graph_wrel, graph_brel, graph_wroot, fc_w, fc_b):
    e = edge_index.shape[1]
    e_pad = ((e + NUM_TILES * CHUNK - 1) // (NUM_TILES * CHUNK)) * (NUM_TILES * CHUNK)
    pad_e = e_pad - e
    pad_rows = N + jnp.arange(pad_e, dtype=jnp.int32) % (N_PAD - N)
    src = jnp.concatenate([edge_index[0], pad_rows])
    dst = jnp.concatenate([edge_index[1], pad_rows])

    w_edge = jnp.concatenate([gcn_w, sage_wl, gin_w, graph_wrel], axis=1)
    w_node = jnp.concatenate([sage_wr, graph_wroot], axis=1)
    b_cat = jnp.concatenate([gcn_b, sage_bl, gin_b, graph_brel]).reshape(1, 64)

    ones128 = jnp.ones((CHUNK, 16), jnp.float32)
    zeros16 = jnp.zeros((N_PAD, 16), jnp.float32)
    zeros64 = jnp.zeros((N_PAD, 64), jnp.float32)

    idx = jnp.stack([src.reshape(-1, CHUNK), dst.reshape(-1, CHUNK)], axis=1)

    cnt2 = _make_sc_hist(e_pad)(idx, ones128, zeros16)

    blk = 1024
    m = pl.pallas_call(
        _tc_project_body,
        grid=(N_PAD // blk,),
        in_specs=[
            pl.BlockSpec((blk, D), lambda i: (i, 0)),
            pl.BlockSpec((D, 64), lambda i: (0, 0)),
            pl.BlockSpec((1, blk, 16), lambda i: (0, i, 0)),
            pl.BlockSpec((1, blk, 16), lambda i: (1, i, 0)),
        ],
        out_specs=pl.BlockSpec((blk, 64), lambda i: (i, 0)),
        out_shape=jax.ShapeDtypeStruct((N_PAD, 64), jnp.float32),
    )(x, w_edge, cnt2, cnt2)

    s2 = _make_sc_segsum(e_pad)(m, idx, zeros64)

    blk2 = 2000
    out = pl.pallas_call(
        _tc_combine_body,
        grid=(N // blk2,),
        in_specs=[
            pl.BlockSpec((blk2, D), lambda i: (i, 0)),
            pl.BlockSpec((1, blk2, 64), lambda i: (0, i, 0)),
            pl.BlockSpec((1, blk2, 64), lambda i: (1, i, 0)),
            pl.BlockSpec((blk2, 64), lambda i: (i, 0)),
            pl.BlockSpec((1, blk2, 16), lambda i: (0, i, 0)),
            pl.BlockSpec((1, blk2, 16), lambda i: (1, i, 0)),
            pl.BlockSpec((D, 32), lambda i: (0, 0)),
            pl.BlockSpec((1, 64), lambda i: (0, 0)),
            pl.BlockSpec((64, OUT), lambda i: (0, 0)),
            pl.BlockSpec((1, OUT), lambda i: (0, 0)),
        ],
        out_specs=pl.BlockSpec((blk2, OUT), lambda i: (i, 0)),
        out_shape=jax.ShapeDtypeStruct((N, OUT), jnp.float32),
    )(x, s2, s2, m, cnt2, cnt2, w_node, b_cat, fc_w,
      fc_b.reshape(1, OUT))
    return out


# speedup vs baseline: 38.6000x; 1.0543x over previous
import functools
import jax
import jax.numpy as jnp
from jax import lax
from jax.experimental import pallas as pl
from jax.experimental.pallas import tpu as pltpu
from jax.experimental.pallas import tpu_sc as plsc

N = 10000
D = 128
H = 16
OUT = 16

N_PAD = 10240
CHUNK = 128
NUM_TILES = 32
ROWS_PER_TILE = N_PAD // 16
K = 4
NBUF = 2


def _tile_id():
    cid = lax.axis_index("c")
    sid = lax.axis_index("s")
    return cid * 16 + sid, cid, sid


def _make_sc_hist(e_pad):
    chunks_per_tile = e_pad // (NUM_TILES * CHUNK)
    mesh = plsc.VectorSubcoreMesh(core_axis_name="c", subcore_axis_name="s")

    @functools.partial(
        pl.kernel,
        out_type=jax.ShapeDtypeStruct((2, N_PAD, 16), jnp.float32),
        mesh=mesh,
        compiler_params=pltpu.CompilerParams(use_tc_tiling_on_sc=False),
        scratch_types=[
            pltpu.VMEM((e_pad // (NUM_TILES * CHUNK), CHUNK), jnp.int32),
            pltpu.VMEM((CHUNK, 16), jnp.float32),
            pltpu.VMEM_SHARED((N_PAD, 16), jnp.float32),
        ],
    )
    def hist(idx_hbm, ones_hbm, zeros_hbm, out_hbm, dst_v, ones_v, cnt_sh):
        tid, cid, sid = _tile_id()
        row0 = sid * ROWS_PER_TILE
        pltpu.sync_copy(zeros_hbm.at[pl.ds(row0, ROWS_PER_TILE)],
                        cnt_sh.at[pl.ds(row0, ROWS_PER_TILE)])
        pltpu.sync_copy(ones_hbm, ones_v)
        pltpu.sync_copy(idx_hbm.at[1, pl.ds(tid * chunks_per_tile, chunks_per_tile)],
                        dst_v)
        plsc.subcore_barrier()

        def body(j, carry):
            pltpu.sync_copy(ones_v, cnt_sh.at[dst_v.at[j]], add=True)
            return carry

        lax.fori_loop(0, chunks_per_tile, body, 0)
        plsc.subcore_barrier()
        pltpu.sync_copy(cnt_sh.at[pl.ds(row0, ROWS_PER_TILE)],
                        out_hbm.at[cid, pl.ds(row0, ROWS_PER_TILE)])

    return hist


def _make_sc_segsum(e_pad):
    chunks_per_tile = e_pad // (NUM_TILES * CHUNK)
    mesh = plsc.VectorSubcoreMesh(core_axis_name="c", subcore_axis_name="s")

    @functools.partial(
        pl.kernel,
        out_type=jax.ShapeDtypeStruct((2, N_PAD, 64), jnp.float32),
        mesh=mesh,
        compiler_params=pltpu.CompilerParams(use_tc_tiling_on_sc=False),
        scratch_types=[
            pltpu.VMEM((chunks_per_tile, CHUNK), jnp.int32),
            pltpu.VMEM((chunks_per_tile, CHUNK), jnp.int32),
            pltpu.VMEM((CHUNK, 64), jnp.float32),
            pltpu.VMEM_SHARED((N_PAD, 64), jnp.float32),
            pltpu.VMEM_SHARED((N_PAD, 64), jnp.float32),
        ],
    )
    def segsum(m_hbm, idx_hbm, zeros_hbm, out_hbm, src_v, dst_v, rows_v,
               m_sh, s_sh):
        tid, cid, sid = _tile_id()
        row0 = sid * ROWS_PER_TILE
        pltpu.sync_copy(zeros_hbm.at[pl.ds(row0, ROWS_PER_TILE)],
                        s_sh.at[pl.ds(row0, ROWS_PER_TILE)])
        pltpu.sync_copy(m_hbm.at[pl.ds(row0, ROWS_PER_TILE)],
                        m_sh.at[pl.ds(row0, ROWS_PER_TILE)])
        pltpu.sync_copy(idx_hbm.at[0, pl.ds(tid * chunks_per_tile, chunks_per_tile)],
                        src_v)
        pltpu.sync_copy(idx_hbm.at[1, pl.ds(tid * chunks_per_tile, chunks_per_tile)],
                        dst_v)
        plsc.subcore_barrier()

        def body(j, carry):
            pltpu.sync_copy(m_sh.at[src_v.at[j]], rows_v)
            pltpu.sync_copy(rows_v, s_sh.at[dst_v.at[j]], add=True)
            return carry

        lax.fori_loop(0, chunks_per_tile, body, 0)
        plsc.subcore_barrier()
        pltpu.sync_copy(s_sh.at[pl.ds(row0, ROWS_PER_TILE)],
                        out_hbm.at[cid, pl.ds(row0, ROWS_PER_TILE)])

    return segsum


def _tc_project_body(x_ref, w_ref, c0_ref, c1_ref, m_ref):
    p = jnp.dot(x_ref[...], w_ref[...], preferred_element_type=jnp.float32)
    deg = c0_ref[0, :, 0:1] + c1_ref[0, :, 0:1] + 1.0
    dinv = lax.rsqrt(deg)
    col = lax.broadcasted_iota(jnp.int32, p.shape, 1)
    m_ref[...] = jnp.where(col < 16, p * dinv, p)


def _tc_combine_body(x_ref, s0_ref, s1_ref, m_ref, c0_ref, c1_ref,
                     wn_ref, bc_ref, fw_ref, fb_ref, out_ref):
    s = s0_ref[0] + s1_ref[0]
    m = m_ref[...]
    cnt = c0_ref[0, :, 0:1] + c1_ref[0, :, 0:1]
    dinv = lax.rsqrt(cnt + 1.0)
    a = jnp.dot(x_ref[...], wn_ref[...], preferred_element_type=jnp.float32)
    bc = bc_ref[...]
    gcn = dinv * (s[:, 0:16] + m[:, 0:16]) + bc[:, 0:16]
    sage = s[:, 16:32] / jnp.maximum(cnt, 1.0) + bc[:, 16:32] + a[:, 0:16]
    gin = m[:, 32:48] + s[:, 32:48] + bc[:, 32:48]
    graph = s[:, 48:64] + bc[:, 48:64] + a[:, 16:32]
    cat = jnp.concatenate(
        [jax.nn.relu(gcn), jax.nn.relu(sage), jax.nn.relu(gin),
         jax.nn.relu(graph)], axis=1)
    out = jnp.dot(cat, fw_ref[...], preferred_element_type=jnp.float32)
    out_ref[...] = jax.nn.sigmoid(out + fb_ref[...])


def kernel(x, edge_index, gcn_w, gcn_b, sage_wl, sage_bl, sage_wr,
           gin_w, gin_b, graph_wrel, graph_brel, graph_wroot, fc_w, fc_b):
    e = edge_index.shape[1]
    e_pad = ((e + NUM_TILES * CHUNK - 1) // (NUM_TILES * CHUNK)) * (NUM_TILES * CHUNK)
    pad_e = e_pad - e
    pad_rows = N + jnp.arange(pad_e, dtype=jnp.int32) % (N_PAD - N)
    eidx = jnp.concatenate(
        [edge_index, jnp.broadcast_to(pad_rows, (2, pad_e))], axis=1)
    idx = eidx.reshape(2, -1, CHUNK)

    w_edge = jnp.concatenate([gcn_w, sage_wl, gin_w, graph_wrel], axis=1)
    w_node = jnp.concatenate([sage_wr, graph_wroot], axis=1)
    b_cat = jnp.concatenate([gcn_b, sage_bl, gin_b, graph_brel]).reshape(1, 64)

    ones128 = jnp.ones((CHUNK, 16), jnp.float32)
    zeros16 = jnp.zeros((N_PAD, 16), jnp.float32)
    zeros64 = jnp.zeros((N_PAD, 64), jnp.float32)

    cnt2 = _make_sc_hist(e_pad)(idx, ones128, zeros16)

    blk = 2048
    m = pl.pallas_call(
        _tc_project_body,
        grid=(N_PAD // blk,),
        in_specs=[
            pl.BlockSpec((blk, D), lambda i: (i, 0)),
            pl.BlockSpec((D, 64), lambda i: (0, 0)),
            pl.BlockSpec((1, blk, 16), lambda i: (0, i, 0)),
            pl.BlockSpec((1, blk, 16), lambda i: (1, i, 0)),
        ],
        out_specs=pl.BlockSpec((blk, 64), lambda i: (i, 0)),
        out_shape=jax.ShapeDtypeStruct((N_PAD, 64), jnp.float32),
    )(x, w_edge, cnt2, cnt2)

    s2 = _make_sc_segsum(e_pad)(m, idx, zeros64)

    blk2 = 2000
    out = pl.pallas_call(
        _tc_combine_body,
        grid=(N // blk2,),
        in_specs=[
            pl.BlockSpec((blk2, D), lambda i: (i, 0)),
            pl.BlockSpec((1, blk2, 64), lambda i: (0, i, 0)),
            pl.BlockSpec((1, blk2, 64), lambda i: (1, i, 0)),
            pl.BlockSpec((blk2, 64), lambda i: (i, 0)),
            pl.BlockSpec((1, blk2, 16), lambda i: (0, i, 0)),
            pl.BlockSpec((1, blk2, 16), lambda i: (1, i, 0)),
            pl.BlockSpec((D, 32), lambda i: (0, 0)),
            pl.BlockSpec((1, 64), lambda i: (0, 0)),
            pl.BlockSpec((64, OUT), lambda i: (0, 0)),
            pl.BlockSpec((1, OUT), lambda i: (0, 0)),
        ],
        out_specs=pl.BlockSpec((blk2, OUT), lambda i: (i, 0)),
        out_shape=jax.ShapeDtypeStruct((N, OUT), jnp.float32),
    )(x, s2, s2, m, cnt2, cnt2, w_node, b_cat, fc_w,
      fc_b.reshape(1, OUT))
    return out
